# baseline (speedup 1.0000x reference)
import functools

import jax
import jax.numpy as jnp
from jax import lax
from jax.experimental import pallas as pl
from jax.experimental.pallas import tpu as pltpu
from jax.experimental.pallas import tpu_sc as plsc

BATCH = 4096
SEQ = 200
D_MODEL = 64
NUM_CORES = 2
NUM_SUBCORES = 16
NW = NUM_CORES * NUM_SUBCORES
ROWS_PW = BATCH // NW
NBUF = 4

_mesh = plsc.VectorSubcoreMesh(core_axis_name="c", subcore_axis_name="s")


@functools.partial(
    pl.kernel,
    mesh=_mesh,
    compiler_params=pltpu.CompilerParams(use_tc_tiling_on_sc=False),
    out_type=jax.ShapeDtypeStruct((BATCH, SEQ, D_MODEL), jnp.float32),
    scratch_types=[
        pltpu.VMEM((ROWS_PW, SEQ), jnp.int32),
        pltpu.VMEM((NBUF, SEQ, D_MODEL), jnp.float32),
        [pltpu.SemaphoreType.DMA] * NBUF,
    ],
)
def _embed_gather(x_hbm, table_hbm, out_hbm, idx_v, rows_v, sems):
    wid = lax.axis_index("s") * NUM_CORES + lax.axis_index("c")
    base_row = wid * ROWS_PW

    pltpu.sync_copy(x_hbm.at[pl.ds(base_row, ROWS_PW)], idx_v)

    for b in range(NBUF):
        pltpu.async_copy(table_hbm.at[idx_v.at[b]], rows_v.at[b], sems[b])

    def body(i, carry):
        g = i * NBUF
        for b in range(NBUF):
            c = g + b
            pltpu.make_async_copy(
                table_hbm.at[idx_v.at[c]], rows_v.at[b], sems[b]
            ).wait()
            pltpu.sync_copy(rows_v.at[b], out_hbm.at[base_row + c])

            @pl.when(c + NBUF < ROWS_PW)
            def _issue():
                pltpu.async_copy(
                    table_hbm.at[idx_v.at[c + NBUF]], rows_v.at[b], sems[b]
                )

        return carry

    lax.fori_loop(0, ROWS_PW // NBUF, body, 0)


def kernel(x, table):
    return _embed_gather(x, table)
